# Initial kernel scaffold; baseline (speedup 1.0000x reference)
import jax
import jax.numpy as jnp
from jax.experimental import pallas as pl


def kernel(adj, features, W1, W2, W_o, bias):
    raise NotImplementedError("write your pallas kernel here")


# speedup vs baseline: 90.0505x; 90.0505x over previous
import functools

import jax
import jax.numpy as jnp
from jax import lax
from jax.experimental import pallas as pl
from jax.experimental.pallas import tpu as pltpu
from jax.experimental.pallas import tpu_sc as plsc

N = 10000
NPAD = 10240
E = 320000
K = 8
D = 8
KD = K * D
ITERATIONS = 5
EPS = 1e-8

NC = 2
NS = 16
NW = NC * NS
EPW = E // NW
CHUNK = 80
NCHUNK = EPW // CHUNK
GROUPS = CHUNK // 16
ROWS_PER_SUB = NPAD // NS
ZB = 128

_BLK = 1000
_NBLK = N // _BLK
_BLKP = 1024
_NBLKP = NPAD // _BLKP


def _edge_body(src_hbm, dst_hbm, z_hbm, c_hbm, part_hbm,
               idx_s, idx_d, zrows, crows, wrows, zbuf, agg,
               sem_z0, sem_z1, sem_c0, sem_c1, sem_w0, sem_w1):
    cid = lax.axis_index("c")
    sid = lax.axis_index("s")
    sem_z = (sem_z0, sem_z1)
    sem_c = (sem_c0, sem_c1)
    sem_w = (sem_w0, sem_w1)
    wid = cid * NS + sid

    pltpu.sync_copy(src_hbm.at[pl.ds(wid * NCHUNK, NCHUNK)], idx_s)
    pltpu.sync_copy(dst_hbm.at[pl.ds(wid * NCHUNK, NCHUNK)], idx_d)

    def _zero_row(i, _):
        for j in range(4):
            zbuf[i, pl.ds(16 * j, 16)] = jnp.zeros((16,), jnp.float32)
        return 0
    lax.fori_loop(0, ZB, _zero_row, 0)
    for r in range(ROWS_PER_SUB // ZB):
        pltpu.sync_copy(zbuf, agg.at[pl.ds(sid * ROWS_PER_SUB + r * ZB, ZB)])
    plsc.subcore_barrier()

    lane = lax.iota(jnp.int32, 16)

    def _start(ch, b):
        pltpu.async_copy(z_hbm.at[idx_s.at[ch]], zrows.at[b], sem_z[b])
        pltpu.async_copy(c_hbm.at[idx_d.at[ch]], crows.at[b], sem_c[b])

    def _drain_w(ch, b):
        pltpu.make_async_copy(wrows.at[b], agg.at[idx_d.at[ch]], sem_w[b]).wait()

    def _process(ch, b, drain):
        pltpu.make_async_copy(z_hbm.at[idx_s.at[ch]], zrows.at[b], sem_z[b]).wait()
        pltpu.make_async_copy(c_hbm.at[idx_d.at[ch]], crows.at[b], sem_c[b]).wait()
        zr = zrows.at[b]
        cr = crows.at[b]
        wr = wrows.at[b]

        def _group(g, _):
            row = g * 16 + lane
            colmod = [(lane + j) & (D - 1) for j in range(D)]
            hi = lane >> 3
            chcol = [(((k + hi) & (K - 1)) * D) for k in range(K)]
            logits = []
            for k in range(K):
                acc = None
                for j in range(D):
                    col = chcol[k] + colmod[j]
                    zv = plsc.load_gather(zr, [row, col])
                    cv = plsc.load_gather(cr, [row, col])
                    acc = zv * cv if acc is None else acc + zv * cv
                logits.append(acc)
            es = [jnp.exp(l) for l in logits]
            s = es[0]
            for k in range(1, K):
                s = s + es[k]
            r = 1.0 / s
            for k in range(K):
                p = es[k] * r
                for j in range(D):
                    col = chcol[k] + colmod[j]
                    zv = plsc.load_gather(zr, [row, col])
                    plsc.store_scatter(wr, [row, col], zv * p)
            return 0
        lax.fori_loop(0, GROUPS, _group, 0)

        pltpu.async_copy(wr, agg.at[idx_d.at[ch]], sem_w[b], add=True)
        if drain is not None:
            _drain_w(ch - 1, 1 - b)

    _start(0, 0)
    _start(1, 1)
    _process(0, 0, None)
    _start(2, 0)

    def _pair(i, _):
        _process(2 * i + 1, 1, True)
        _start(2 * i + 3, 1)
        _process(2 * i + 2, 0, True)
        _start(2 * i + 4, 0)
        return 0

    lax.fori_loop(0, (NCHUNK - 5) // 2, _pair, 0)
    _process(NCHUNK - 4, 1, True)
    _start(NCHUNK - 2, 1)
    _process(NCHUNK - 3, 0, True)
    _start(NCHUNK - 1, 0)
    _process(NCHUNK - 2, 1, True)
    _process(NCHUNK - 1, 0, True)
    _drain_w(NCHUNK - 1, 0)
    plsc.subcore_barrier()

    off = sid * ROWS_PER_SUB
    pltpu.sync_copy(agg.at[pl.ds(off, ROWS_PER_SUB)],
                    part_hbm.at[cid, pl.ds(off, ROWS_PER_SUB)])


_edge_call = functools.partial(
    pl.kernel,
    out_type=jax.ShapeDtypeStruct((NC, NPAD, KD), jnp.float32),
    mesh=plsc.VectorSubcoreMesh(core_axis_name="c", subcore_axis_name="s"),
    scratch_types=[
        pltpu.VMEM((NCHUNK, CHUNK), jnp.int32),
        pltpu.VMEM((NCHUNK, CHUNK), jnp.int32),
        pltpu.VMEM((2, CHUNK, KD), jnp.float32),
        pltpu.VMEM((2, CHUNK, KD), jnp.float32),
        pltpu.VMEM((2, CHUNK, KD), jnp.float32),
        pltpu.VMEM((ZB, KD), jnp.float32),
        pltpu.VMEM_SHARED((NPAD, KD), jnp.float32),
        pltpu.SemaphoreType.DMA,
        pltpu.SemaphoreType.DMA,
        pltpu.SemaphoreType.DMA,
        pltpu.SemaphoreType.DMA,
        pltpu.SemaphoreType.DMA,
        pltpu.SemaphoreType.DMA,
    ],
    compiler_params=pltpu.CompilerParams(
        needs_layout_passes=False, use_tc_tiling_on_sc=False
    ),
)(_edge_body)


def _edge(src, dst, z, c):
    return _edge_call(src, dst, z, c)


def _group_sum_mat():
    rr = lax.broadcasted_iota(jnp.int32, (KD, KD), 0) // D
    cc = lax.broadcasted_iota(jnp.int32, (KD, KD), 1) // D
    return (rr == cc).astype(jnp.float32)


def _prep_body(h_ref, w_ref, o_ref):
    z = jnp.dot(h_ref[...], w_ref[...], preferred_element_type=jnp.float32)
    s = jnp.dot(z * z, _group_sum_mat(), preferred_element_type=jnp.float32)
    o_ref[...] = z / (jnp.sqrt(s) + EPS)


def _prep(h, w):
    hd = h.shape[1]
    return pl.pallas_call(
        _prep_body,
        grid=(_NBLK,),
        in_specs=[
            pl.BlockSpec((_BLK, hd), lambda i: (i, 0)),
            pl.BlockSpec((hd, KD), lambda i: (0, 0)),
        ],
        out_specs=pl.BlockSpec((_BLK, KD), lambda i: (i, 0)),
        out_shape=jax.ShapeDtypeStruct((N, KD), jnp.float32),
    )(h, w)


def _combine_body(z_ref, p0_ref, p1_ref, o_ref):
    v = z_ref[...] + p0_ref[...] + p1_ref[...]
    s = jnp.dot(v * v, _group_sum_mat(), preferred_element_type=jnp.float32)
    o_ref[...] = v / (jnp.sqrt(s) + EPS)


def _combine(z, p0, p1):
    return pl.pallas_call(
        _combine_body,
        grid=(_NBLKP,),
        in_specs=[pl.BlockSpec((_BLKP, KD), lambda i: (i, 0))] * 3,
        out_specs=pl.BlockSpec((_BLKP, KD), lambda i: (i, 0)),
        out_shape=jax.ShapeDtypeStruct((NPAD, KD), jnp.float32),
    )(z, p0, p1)


def _final_body(h_ref, w_ref, b_ref, o_ref):
    o_ref[...] = (
        jnp.dot(h_ref[...], w_ref[...], preferred_element_type=jnp.float32)
        + b_ref[...]
    )


def _final(h, w_o, bias):
    od = w_o.shape[1]
    return pl.pallas_call(
        _final_body,
        grid=(_NBLK,),
        in_specs=[
            pl.BlockSpec((_BLK, KD), lambda i: (i, 0)),
            pl.BlockSpec((KD, od), lambda i: (0, 0)),
            pl.BlockSpec((1, od), lambda i: (0, 0)),
        ],
        out_specs=pl.BlockSpec((_BLK, od), lambda i: (i, 0)),
        out_shape=jax.ShapeDtypeStruct((N, od), jnp.float32),
    )(h, w_o, bias)


def _disconv(src, dst, h, w):
    z = jnp.pad(_prep(h, w), ((0, NPAD - N), (0, 0)))
    c = z
    for _ in range(ITERATIONS):
        parts = _edge(src, dst, z, c)
        c = _combine(z, parts[0], parts[1])
    return c[:N]


def kernel(adj, features, W1, W2, W_o, bias):
    src = adj[0].reshape(E // CHUNK, CHUNK)
    dst = adj[1].reshape(E // CHUNK, CHUNK)
    h = _disconv(src, dst, features, W1)
    h = _disconv(src, dst, h, W2)
    return _final(h, W_o, bias)


# speedup vs baseline: 103.8935x; 1.1537x over previous
import functools

import jax
import jax.numpy as jnp
from jax import lax
from jax.experimental import pallas as pl
from jax.experimental.pallas import tpu as pltpu
from jax.experimental.pallas import tpu_sc as plsc

N = 10000
NPAD = 10240
E = 320000
K = 8
D = 8
KD = K * D
ITERATIONS = 5
EPS = 1e-8

NC = 2
NS = 16
NW = NC * NS
EPW = E // NW
CHUNK = 80
NCHUNK = EPW // CHUNK
GROUPS = CHUNK // 16
ROWS_PER_SUB = NPAD // NS
ZB = 128

_BLK = 1000
_NBLK = N // _BLK
_BLKP = 1024
_NBLKP = NPAD // _BLKP


def _edge_body(src_hbm, dst_hbm, z_hbm, c_hbm, part_hbm,
               idx_s, idx_d, zrows, crows, wrows, zbuf, ebuf, agg,
               sem_z0, sem_z1, sem_c0, sem_c1, sem_w0, sem_w1):
    cid = lax.axis_index("c")
    sid = lax.axis_index("s")
    sem_z = (sem_z0, sem_z1)
    sem_c = (sem_c0, sem_c1)
    sem_w = (sem_w0, sem_w1)
    wid = cid * NS + sid

    pltpu.sync_copy(src_hbm.at[pl.ds(wid * NCHUNK, NCHUNK)], idx_s)
    pltpu.sync_copy(dst_hbm.at[pl.ds(wid * NCHUNK, NCHUNK)], idx_d)

    def _zero_row(i, _):
        for j in range(4):
            zbuf[i, pl.ds(16 * j, 16)] = jnp.zeros((16,), jnp.float32)
        return 0
    lax.fori_loop(0, ZB, _zero_row, 0)
    for r in range(ROWS_PER_SUB // ZB):
        pltpu.sync_copy(zbuf, agg.at[pl.ds(sid * ROWS_PER_SUB + r * ZB, ZB)])
    plsc.subcore_barrier()

    lane = lax.iota(jnp.int32, 16)

    def _start(ch, b):
        pltpu.async_copy(z_hbm.at[idx_s.at[ch]], zrows.at[b], sem_z[b])
        pltpu.async_copy(c_hbm.at[idx_d.at[ch]], crows.at[b], sem_c[b])

    def _drain_w(ch, b):
        pltpu.make_async_copy(wrows.at[b], agg.at[idx_d.at[ch]], sem_w[b]).wait()

    def _process(ch, b, drain):
        pltpu.make_async_copy(z_hbm.at[idx_s.at[ch]], zrows.at[b], sem_z[b]).wait()
        pltpu.make_async_copy(c_hbm.at[idx_d.at[ch]], crows.at[b], sem_c[b]).wait()
        zr = zrows.at[b]
        cr = crows.at[b]
        wr = wrows.at[b]

        def _group(g, _):
            row = g * 16 + lane
            colmod = [(lane + j) & (D - 1) for j in range(D)]
            hi = lane >> 3

            def _chan(k, s):
                chc = ((k + hi) & (K - 1)) * D
                acc = None
                for j in range(D):
                    col = chc + colmod[j]
                    zv = plsc.load_gather(zr, [row, col])
                    cv = plsc.load_gather(cr, [row, col])
                    acc = zv * cv if acc is None else acc + zv * cv
                e = jnp.exp(acc)
                ebuf[pl.ds(k * 16, 16)] = e
                return s + e
            s = lax.fori_loop(0, K, _chan, jnp.zeros((16,), jnp.float32),
                              unroll=2)
            r = 1.0 / s

            def _wchan(k, _):
                chc = ((k + hi) & (K - 1)) * D
                p = ebuf[pl.ds(k * 16, 16)] * r
                for j in range(D):
                    col = chc + colmod[j]
                    zv = plsc.load_gather(zr, [row, col])
                    plsc.store_scatter(wr, [row, col], zv * p)
                return 0
            lax.fori_loop(0, K, _wchan, 0, unroll=2)
            return 0
        lax.fori_loop(0, GROUPS, _group, 0)

        pltpu.async_copy(wr, agg.at[idx_d.at[ch]], sem_w[b], add=True)
        if drain is not None:
            _drain_w(ch - 1, 1 - b)

    _start(0, 0)
    _start(1, 1)
    _process(0, 0, None)
    _start(2, 0)

    def _pair(i, _):
        _process(2 * i + 1, 1, True)
        _start(2 * i + 3, 1)
        _process(2 * i + 2, 0, True)
        _start(2 * i + 4, 0)
        return 0

    lax.fori_loop(0, (NCHUNK - 5) // 2, _pair, 0)
    _process(NCHUNK - 4, 1, True)
    _start(NCHUNK - 2, 1)
    _process(NCHUNK - 3, 0, True)
    _start(NCHUNK - 1, 0)
    _process(NCHUNK - 2, 1, True)
    _process(NCHUNK - 1, 0, True)
    _drain_w(NCHUNK - 1, 0)
    plsc.subcore_barrier()

    off = sid * ROWS_PER_SUB
    pltpu.sync_copy(agg.at[pl.ds(off, ROWS_PER_SUB)],
                    part_hbm.at[cid, pl.ds(off, ROWS_PER_SUB)])


_edge_call = functools.partial(
    pl.kernel,
    out_type=jax.ShapeDtypeStruct((NC, NPAD, KD), jnp.float32),
    mesh=plsc.VectorSubcoreMesh(core_axis_name="c", subcore_axis_name="s"),
    scratch_types=[
        pltpu.VMEM((NCHUNK, CHUNK), jnp.int32),
        pltpu.VMEM((NCHUNK, CHUNK), jnp.int32),
        pltpu.VMEM((2, CHUNK, KD), jnp.float32),
        pltpu.VMEM((2, CHUNK, KD), jnp.float32),
        pltpu.VMEM((2, CHUNK, KD), jnp.float32),
        pltpu.VMEM((ZB, KD), jnp.float32),
        pltpu.VMEM((K * 16,), jnp.float32),
        pltpu.VMEM_SHARED((NPAD, KD), jnp.float32),
        pltpu.SemaphoreType.DMA,
        pltpu.SemaphoreType.DMA,
        pltpu.SemaphoreType.DMA,
        pltpu.SemaphoreType.DMA,
        pltpu.SemaphoreType.DMA,
        pltpu.SemaphoreType.DMA,
    ],
    compiler_params=pltpu.CompilerParams(
        needs_layout_passes=False, use_tc_tiling_on_sc=False
    ),
)(_edge_body)


def _edge(src, dst, z, c):
    return _edge_call(src, dst, z, c)


def _group_sum_mat():
    rr = lax.broadcasted_iota(jnp.int32, (KD, KD), 0) // D
    cc = lax.broadcasted_iota(jnp.int32, (KD, KD), 1) // D
    return (rr == cc).astype(jnp.float32)


def _prep_body(h_ref, w_ref, o_ref):
    z = jnp.dot(h_ref[...], w_ref[...], preferred_element_type=jnp.float32)
    s = jnp.dot(z * z, _group_sum_mat(), preferred_element_type=jnp.float32)
    o_ref[...] = z / (jnp.sqrt(s) + EPS)


def _prep(h, w):
    hd = h.shape[1]
    return pl.pallas_call(
        _prep_body,
        grid=(_NBLK,),
        in_specs=[
            pl.BlockSpec((_BLK, hd), lambda i: (i, 0)),
            pl.BlockSpec((hd, KD), lambda i: (0, 0)),
        ],
        out_specs=pl.BlockSpec((_BLK, KD), lambda i: (i, 0)),
        out_shape=jax.ShapeDtypeStruct((N, KD), jnp.float32),
    )(h, w)


def _combine_body(z_ref, p0_ref, p1_ref, o_ref):
    v = z_ref[...] + p0_ref[...] + p1_ref[...]
    s = jnp.dot(v * v, _group_sum_mat(), preferred_element_type=jnp.float32)
    o_ref[...] = v / (jnp.sqrt(s) + EPS)


def _combine(z, p0, p1):
    return pl.pallas_call(
        _combine_body,
        grid=(_NBLKP,),
        in_specs=[pl.BlockSpec((_BLKP, KD), lambda i: (i, 0))] * 3,
        out_specs=pl.BlockSpec((_BLKP, KD), lambda i: (i, 0)),
        out_shape=jax.ShapeDtypeStruct((NPAD, KD), jnp.float32),
    )(z, p0, p1)


def _final_body(h_ref, w_ref, b_ref, o_ref):
    o_ref[...] = (
        jnp.dot(h_ref[...], w_ref[...], preferred_element_type=jnp.float32)
        + b_ref[...]
    )


def _final(h, w_o, bias):
    od = w_o.shape[1]
    return pl.pallas_call(
        _final_body,
        grid=(_NBLK,),
        in_specs=[
            pl.BlockSpec((_BLK, KD), lambda i: (i, 0)),
            pl.BlockSpec((KD, od), lambda i: (0, 0)),
            pl.BlockSpec((1, od), lambda i: (0, 0)),
        ],
        out_specs=pl.BlockSpec((_BLK, od), lambda i: (i, 0)),
        out_shape=jax.ShapeDtypeStruct((N, od), jnp.float32),
    )(h, w_o, bias)


def _disconv(src, dst, h, w):
    z = jnp.pad(_prep(h, w), ((0, NPAD - N), (0, 0)))
    c = z
    for _ in range(ITERATIONS):
        parts = _edge(src, dst, z, c)
        c = _combine(z, parts[0], parts[1])
    return c[:N]


def kernel(adj, features, W1, W2, W_o, bias):
    src = adj[0].reshape(E // CHUNK, CHUNK)
    dst = adj[1].reshape(E // CHUNK, CHUNK)
    h = _disconv(src, dst, features, W1)
    h = _disconv(src, dst, h, W2)
    return _final(h, W_o, bias)


# speedup vs baseline: 104.6486x; 1.0073x over previous
import functools

import jax
import jax.numpy as jnp
from jax import lax
from jax.experimental import pallas as pl
from jax.experimental.pallas import tpu as pltpu
from jax.experimental.pallas import tpu_sc as plsc

N = 10000
NPAD = 10240
E = 320000
K = 8
D = 8
KD = K * D
ITERATIONS = 5
EPS = 1e-8

NC = 2
NS = 16
NW = NC * NS
EPW = E // NW
CHUNK = 80
NCHUNK = EPW // CHUNK
GROUPS = CHUNK // 16
ROWS_PER_SUB = NPAD // NS
ZB = 128

_BLK = 1000
_NBLK = N // _BLK
_BLKP = 1024
_NBLKP = NPAD // _BLKP


def _edge_body(src_hbm, dst_hbm, z_hbm, c_hbm, part_hbm,
               idx_s, idx_d, zrows, crows, wrows, zbuf, ebuf, agg,
               sem_z0, sem_z1, sem_c0, sem_c1, sem_w0, sem_w1):
    cid = lax.axis_index("c")
    sid = lax.axis_index("s")
    sem_z = (sem_z0, sem_z1)
    sem_c = (sem_c0, sem_c1)
    sem_w = (sem_w0, sem_w1)
    wid = cid * NS + sid

    pltpu.sync_copy(src_hbm.at[pl.ds(wid * NCHUNK, NCHUNK)], idx_s)
    pltpu.sync_copy(dst_hbm.at[pl.ds(wid * NCHUNK, NCHUNK)], idx_d)

    def _zero_row(i, _):
        for j in range(4):
            zbuf[i, pl.ds(16 * j, 16)] = jnp.zeros((16,), jnp.float32)
        return 0
    lax.fori_loop(0, ZB, _zero_row, 0)
    for r in range(ROWS_PER_SUB // ZB):
        pltpu.sync_copy(zbuf, agg.at[pl.ds(sid * ROWS_PER_SUB + r * ZB, ZB)])
    plsc.subcore_barrier()

    lane = lax.iota(jnp.int32, 16)

    def _start(ch, b):
        pltpu.async_copy(z_hbm.at[idx_s.at[ch]], zrows.at[b], sem_z[b])
        pltpu.async_copy(c_hbm.at[idx_d.at[ch]], crows.at[b], sem_c[b])

    def _drain_w(ch, b):
        pltpu.make_async_copy(wrows.at[b], agg.at[idx_d.at[ch]], sem_w[b]).wait()

    def _process(ch, b, drain):
        pltpu.make_async_copy(z_hbm.at[idx_s.at[ch]], zrows.at[b], sem_z[b]).wait()
        pltpu.make_async_copy(c_hbm.at[idx_d.at[ch]], crows.at[b], sem_c[b]).wait()
        zr = zrows.at[b]
        cr = crows.at[b]
        wr = wrows.at[b]

        def _group(g, _):
            row = g * 16 + lane
            colmod = [(lane + j) & (D - 1) for j in range(D)]
            hi = lane >> 3

            def _chan(k, s):
                chc = ((k + hi) & (K - 1)) * D
                acc = None
                for j in range(D):
                    col = chc + colmod[j]
                    zv = plsc.load_gather(zr, [row, col])
                    cv = plsc.load_gather(cr, [row, col])
                    acc = zv * cv if acc is None else acc + zv * cv
                e = jnp.exp(acc)
                ebuf[pl.ds(k * 16, 16)] = e
                return s + e
            s = lax.fori_loop(0, K, _chan, jnp.zeros((16,), jnp.float32),
                              unroll=2)
            r = 1.0 / s

            def _wchan(k, _):
                chc = ((k + hi) & (K - 1)) * D
                p = ebuf[pl.ds(k * 16, 16)] * r
                for j in range(D):
                    col = chc + colmod[j]
                    zv = plsc.load_gather(zr, [row, col])
                    plsc.store_scatter(wr, [row, col], zv * p)
                return 0
            lax.fori_loop(0, K, _wchan, 0, unroll=4)
            return 0
        lax.fori_loop(0, GROUPS, _group, 0)

        pltpu.async_copy(wr, agg.at[idx_d.at[ch]], sem_w[b], add=True)
        if drain is not None:
            _drain_w(ch - 1, 1 - b)

    _start(0, 0)
    _start(1, 1)
    _process(0, 0, None)
    _start(2, 0)

    def _pair(i, _):
        _process(2 * i + 1, 1, True)
        _start(2 * i + 3, 1)
        _process(2 * i + 2, 0, True)
        _start(2 * i + 4, 0)
        return 0

    lax.fori_loop(0, (NCHUNK - 5) // 2, _pair, 0)
    _process(NCHUNK - 4, 1, True)
    _start(NCHUNK - 2, 1)
    _process(NCHUNK - 3, 0, True)
    _start(NCHUNK - 1, 0)
    _process(NCHUNK - 2, 1, True)
    _process(NCHUNK - 1, 0, True)
    _drain_w(NCHUNK - 1, 0)
    plsc.subcore_barrier()

    off = sid * ROWS_PER_SUB
    pltpu.sync_copy(agg.at[pl.ds(off, ROWS_PER_SUB)],
                    part_hbm.at[cid, pl.ds(off, ROWS_PER_SUB)])


_edge_call = functools.partial(
    pl.kernel,
    out_type=jax.ShapeDtypeStruct((NC, NPAD, KD), jnp.float32),
    mesh=plsc.VectorSubcoreMesh(core_axis_name="c", subcore_axis_name="s"),
    scratch_types=[
        pltpu.VMEM((NCHUNK, CHUNK), jnp.int32),
        pltpu.VMEM((NCHUNK, CHUNK), jnp.int32),
        pltpu.VMEM((2, CHUNK, KD), jnp.float32),
        pltpu.VMEM((2, CHUNK, KD), jnp.float32),
        pltpu.VMEM((2, CHUNK, KD), jnp.float32),
        pltpu.VMEM((ZB, KD), jnp.float32),
        pltpu.VMEM((K * 16,), jnp.float32),
        pltpu.VMEM_SHARED((NPAD, KD), jnp.float32),
        pltpu.SemaphoreType.DMA,
        pltpu.SemaphoreType.DMA,
        pltpu.SemaphoreType.DMA,
        pltpu.SemaphoreType.DMA,
        pltpu.SemaphoreType.DMA,
        pltpu.SemaphoreType.DMA,
    ],
    compiler_params=pltpu.CompilerParams(
        needs_layout_passes=False, use_tc_tiling_on_sc=False
    ),
)(_edge_body)


def _edge(src, dst, z, c):
    return _edge_call(src, dst, z, c)


def _group_sum_mat():
    rr = lax.broadcasted_iota(jnp.int32, (KD, KD), 0) // D
    cc = lax.broadcasted_iota(jnp.int32, (KD, KD), 1) // D
    return (rr == cc).astype(jnp.float32)


def _prep_body(h_ref, w_ref, o_ref):
    z = jnp.dot(h_ref[...], w_ref[...], preferred_element_type=jnp.float32)
    s = jnp.dot(z * z, _group_sum_mat(), preferred_element_type=jnp.float32)
    o_ref[...] = z / (jnp.sqrt(s) + EPS)


def _prep(h, w):
    hd = h.shape[1]
    return pl.pallas_call(
        _prep_body,
        grid=(_NBLKP,),
        in_specs=[
            pl.BlockSpec((_BLKP, hd), lambda i: (i, 0)),
            pl.BlockSpec((hd, KD), lambda i: (0, 0)),
        ],
        out_specs=pl.BlockSpec((_BLKP, KD), lambda i: (i, 0)),
        out_shape=jax.ShapeDtypeStruct((NPAD, KD), jnp.float32),
    )(h, w)


def _combine_body(z_ref, p0_ref, p1_ref, o_ref):
    v = z_ref[...] + p0_ref[...] + p1_ref[...]
    s = jnp.dot(v * v, _group_sum_mat(), preferred_element_type=jnp.float32)
    o_ref[...] = v / (jnp.sqrt(s) + EPS)


def _combine(z, p0, p1):
    return pl.pallas_call(
        _combine_body,
        grid=(_NBLKP,),
        in_specs=[pl.BlockSpec((_BLKP, KD), lambda i: (i, 0))] * 3,
        out_specs=pl.BlockSpec((_BLKP, KD), lambda i: (i, 0)),
        out_shape=jax.ShapeDtypeStruct((NPAD, KD), jnp.float32),
    )(z, p0, p1)


def _final_body(h_ref, w_ref, b_ref, o_ref):
    o_ref[...] = (
        jnp.dot(h_ref[...], w_ref[...], preferred_element_type=jnp.float32)
        + b_ref[...]
    )


def _final(h, w_o, bias):
    od = w_o.shape[1]
    return pl.pallas_call(
        _final_body,
        grid=(_NBLKP,),
        in_specs=[
            pl.BlockSpec((_BLKP, KD), lambda i: (i, 0)),
            pl.BlockSpec((KD, od), lambda i: (0, 0)),
            pl.BlockSpec((1, od), lambda i: (0, 0)),
        ],
        out_specs=pl.BlockSpec((_BLKP, od), lambda i: (i, 0)),
        out_shape=jax.ShapeDtypeStruct((NPAD, od), jnp.float32),
    )(h, w_o, bias)


def _disconv(src, dst, h, w):
    z = _prep(h, w)
    c = z
    for _ in range(ITERATIONS):
        parts = _edge(src, dst, z, c)
        c = _combine(z, parts[0], parts[1])
    return c


def kernel(adj, features, W1, W2, W_o, bias):
    src = adj[0].reshape(E // CHUNK, CHUNK)
    dst = adj[1].reshape(E // CHUNK, CHUNK)
    h = jnp.pad(features, ((0, NPAD - N), (0, 0)))
    h = _disconv(src, dst, h, W1)
    h = _disconv(src, dst, h, W2)
    return _final(h, W_o, bias)[:N]
